# Initial kernel scaffold; baseline (speedup 1.0000x reference)
import jax
import jax.numpy as jnp
from jax.experimental import pallas as pl


def kernel(boxes, positive_indices, feature_maps_0, feature_maps_1, feature_maps_2, config):
    raise NotImplementedError("write your pallas kernel here")


# speedup vs baseline: 10.9838x; 10.9838x over previous
import functools

import jax
import jax.numpy as jnp
from jax import lax
from jax.experimental import pallas as pl
from jax.experimental.pallas import tpu as pltpu
from jax.experimental.pallas import tpu_sc as plsc

POOLN = 7
PTS = POOLN * POOLN
PADC = 104
NWORK = 32


def _prep(boxes, positive_indices, shapes):
    (h0, w0), (h1, w1), (h2, w2) = shapes
    B, N = boxes.shape[0], boxes.shape[1]
    nbox = B * N
    fb = boxes.reshape(-1, 4)
    y1, x1, y2, x2 = fb[:, 0], fb[:, 1], fb[:, 2], fb[:, 3]
    h = y2 - y1
    w = x2 - x1
    roi_level = jnp.log(h * w) / jnp.log(2.0)
    lvl = jnp.minimum(5, jnp.maximum(3, jnp.ceil(5.0 + roi_level).astype(jnp.int32)))
    li = lvl - 3

    hm1 = jnp.array([h0 - 1, h1 - 1, h2 - 1], jnp.float32)[li]
    wm1 = jnp.array([w0 - 1, w1 - 1, w2 - 1], jnp.float32)[li]
    p = jnp.arange(POOLN, dtype=jnp.float32)
    in_y = y1[:, None] * hm1[:, None] + p[None, :] * (h * hm1 / (POOLN - 1))[:, None]
    in_x = x1[:, None] * wm1[:, None] + p[None, :] * (w * wm1 / (POOLN - 1))[:, None]
    top = jnp.floor(in_y)
    left = jnp.floor(in_x)
    t = jnp.clip(top, 0, hm1[:, None]).astype(jnp.int32)
    btm = jnp.clip(top + 1.0, 0, hm1[:, None]).astype(jnp.int32)
    lft = jnp.clip(left, 0, wm1[:, None]).astype(jnp.int32)
    rgt = jnp.clip(left + 1.0, 0, wm1[:, None]).astype(jnp.int32)
    yl = in_y - top
    xl = in_x - left
    vy = ((in_y >= 0) & (in_y <= hm1[:, None])).astype(jnp.float32)
    vx = ((in_x >= 0) & (in_x <= wm1[:, None])).astype(jnp.float32)
    pos = (positive_indices.reshape(-1) == 1).astype(jnp.float32)
    m = pos[:, None, None] * (vy[:, :, None] * vx[:, None, :])

    wtl = m * ((1.0 - yl)[:, :, None] * (1.0 - xl)[:, None, :])
    wtr = m * ((1.0 - yl)[:, :, None] * xl[:, None, :])
    wbl = m * (yl[:, :, None] * (1.0 - xl)[:, None, :])
    wbr = m * (yl[:, :, None] * xl[:, None, :])

    Wl = jnp.array([w0, w1, w2], jnp.int32)[li]
    HWl = jnp.array([h0 * w0, h1 * w1, h2 * w2], jnp.int32)[li]
    base = jnp.array([0, B * h0 * w0, B * (h0 * w0 + h1 * w1)], jnp.int32)[li]
    bi = jnp.arange(nbox, dtype=jnp.int32) // N
    base_b = base + bi * HWl
    iy_t = t * Wl[:, None]
    iy_b = btm * Wl[:, None]
    itl = base_b[:, None, None] + iy_t[:, :, None] + lft[:, None, :]
    itr = base_b[:, None, None] + iy_t[:, :, None] + rgt[:, None, :]
    ibl = base_b[:, None, None] + iy_b[:, :, None] + lft[:, None, :]
    ibr = base_b[:, None, None] + iy_b[:, :, None] + rgt[:, None, :]

    def pack(a, b):
        z = jnp.stack([a, b], axis=-1).reshape(nbox, 2 * PTS)
        return jnp.pad(z, ((0, 0), (0, PADC - 2 * PTS)))

    idx_all = jnp.concatenate([pack(itl, itr), pack(ibl, ibr)], axis=1)
    w_all = jnp.concatenate([pack(wtl, wtr), pack(wbl, wbr)], axis=1)
    return idx_all.astype(jnp.int32), w_all.astype(jnp.float32)


def _sc_pool(table, idx_all, w_all, nbox, C):
    nbox_pad = idx_all.shape[0] // (2 * PADC)
    steps = nbox_pad // NWORK
    row = 2 * PADC
    mesh = plsc.VectorSubcoreMesh(core_axis_name="c", subcore_axis_name="s",
                                  num_cores=2, num_subcores=16)

    @functools.partial(
        pl.kernel,
        out_type=jax.ShapeDtypeStruct((nbox * PTS * C,), jnp.float32),
        mesh=mesh,
        scratch_types=[
            pltpu.VMEM((row,), jnp.int32),
            pltpu.VMEM((row,), jnp.float32),
            pltpu.VMEM((PADC, C), jnp.float32),
            pltpu.VMEM((PADC, C), jnp.float32),
            pltpu.VMEM((PTS * C,), jnp.float32),
            pltpu.SemaphoreType.DMA,
            pltpu.SemaphoreType.DMA,
        ],
        compiler_params=pltpu.CompilerParams(needs_layout_passes=False),
    )
    def body(idx_hbm, w_hbm, table_hbm, out_hbm, idx_v, w_v, rows_a, rows_b,
             out_v, sem_a, sem_b):
        wid = lax.axis_index("s") * 2 + lax.axis_index("c")

        def box_step(j, carry):
            box = j * NWORK + wid

            @pl.when(box < nbox)
            def _():
                pltpu.sync_copy(idx_hbm.at[pl.ds(box * row, row)], idx_v)
                pltpu.sync_copy(w_hbm.at[pl.ds(box * row, row)], w_v)
                cp_a = pltpu.async_copy(
                    table_hbm.at[idx_v.at[pl.ds(0, PADC)]], rows_a, sem_a)
                cp_b = pltpu.async_copy(
                    table_hbm.at[idx_v.at[pl.ds(PADC, PADC)]], rows_b, sem_b)
                cp_a.wait()
                cp_b.wait()

                def pt_step(p, c2):
                    i0 = jnp.full((16,), 2 * p, jnp.int32)
                    i1 = jnp.full((16,), 2 * p + 1, jnp.int32)
                    wtl = plsc.load_gather(w_v, [i0])
                    wtr = plsc.load_gather(w_v, [i1])
                    wbl = plsc.load_gather(w_v, [i0 + PADC])
                    wbr = plsc.load_gather(w_v, [i1 + PADC])
                    for k in range(C // 16):
                        s = pl.ds(k * 16, 16)
                        acc = (rows_a[2 * p, s] * wtl + rows_a[2 * p + 1, s] * wtr
                               + rows_b[2 * p, s] * wbl + rows_b[2 * p + 1, s] * wbr)
                        out_v[pl.ds(p * C + k * 16, 16)] = acc
                    return c2

                lax.fori_loop(0, PTS, pt_step, 0)
                pltpu.sync_copy(out_v, out_hbm.at[pl.ds(box * PTS * C, PTS * C)])

            return carry

        lax.fori_loop(0, steps, box_step, 0)

    return body(idx_all, w_all, table)


def kernel(boxes, positive_indices, feature_maps_0, feature_maps_1,
           feature_maps_2, config):
    B, N = boxes.shape[0], boxes.shape[1]
    C = feature_maps_0.shape[-1]
    nbox = B * N
    shapes = [(f.shape[1], f.shape[2]) for f in
              (feature_maps_0, feature_maps_1, feature_maps_2)]
    idx_all, w_all = _prep(boxes, positive_indices, shapes)
    nbox_pad = ((nbox + NWORK - 1) // NWORK) * NWORK
    idx_all = jnp.pad(idx_all, ((0, nbox_pad - nbox), (0, 0))).reshape(-1)
    w_all = jnp.pad(w_all, ((0, nbox_pad - nbox), (0, 0))).reshape(-1)
    table = jnp.concatenate([feature_maps_0.reshape(-1, C),
                             feature_maps_1.reshape(-1, C),
                             feature_maps_2.reshape(-1, C)], axis=0)
    out = _sc_pool(table, idx_all, w_all, nbox, C)
    return out.reshape(B, N, POOLN, POOLN, C)
